# baseline (speedup 1.0000x reference)
import functools

import jax
import jax.numpy as jnp
from jax import lax
from jax.experimental import pallas as pl
from jax.experimental.pallas import tpu as pltpu
from jax.experimental.pallas import tpu_sc as plsc

_N = 10000
_E = 160000
_DF = 128
_DE = 16
_D = 64
_HD = 32
_G = 16

_B = 128
_EP = 163840
_EPT = _EP // 16
_NB = _EPT // _B
_NP = 10112
_RPT = _NP // 16

_HP = jax.lax.Precision.HIGHEST
_NODE_BLK = 2000
_EDGE_BLK = 2048
_NBLK = _N // _NODE_BLK


def _dot(a, b):
    return jax.lax.dot(a, b, precision=_HP)


def _log16(u):
    bits = lax.bitcast_convert_type(u, jnp.int32)
    ex = ((bits >> 23) & 0xFF) - 127
    m = lax.bitcast_convert_type((bits & 0x007FFFFF) | 0x3F800000, jnp.float32)
    big = m >= 1.5
    m = jnp.where(big, m * 0.5, m)
    exf = ex.astype(jnp.float32) + jnp.where(big, 1.0, 0.0)
    s = (m - 1.0) / (m + 1.0)
    s2 = s * s
    p = s * (2.0 + s2 * (2.0 / 3.0 + s2 * (2.0 / 5.0 + s2 * (2.0 / 7.0))))
    return exf * 0.6931471805599453 + p


def _sc_edge_body(pa_hbm, qa_hbm, cea_hbm, pb_hbm, qb_hbm, ceb_hbm,
                  srcg_hbm, dstg_hbm, dsts_hbm, eid_hbm,
                  out_hbm, srcg_v, dstg_v, dsts_v, eid_v, p_v, q_v, ce_v,
                  msg_v, zer_v, acc, sem0, sem1):
    cid = lax.axis_index("c")
    sid = lax.axis_index("s")

    zrow = jnp.zeros((16,), jnp.float32)
    tables = [(pa_hbm, qa_hbm, cea_hbm), (pb_hbm, qb_hbm, ceb_hbm)]
    sems = [sem0, sem1]

    for half in range(2):
        p_hbm, q_hbm, ce_hbm = tables[half]

        def zbody(i, _):
            r = i // 2
            c = (i % 2) * 16
            zer_v[r, pl.ds(c, 16)] = zrow
            return 0

        lax.fori_loop(0, _RPT * 2, zbody, 0)
        pltpu.sync_copy(zer_v, acc.at[pl.ds(sid * _RPT, _RPT)])
        plsc.subcore_barrier()

        def load_and_fire(b, k):
            base = cid * _EP + sid * _EPT + b * _B
            pltpu.sync_copy(srcg_hbm.at[pl.ds(base, _B)], srcg_v.at[k])
            pltpu.sync_copy(dstg_hbm.at[pl.ds(base, _B)], dstg_v.at[k])
            pltpu.sync_copy(dsts_hbm.at[pl.ds(base, _B)], dsts_v.at[k])
            pltpu.sync_copy(eid_hbm.at[pl.ds(base, _B)], eid_v.at[k])
            pltpu.async_copy(p_hbm.at[srcg_v.at[k]], p_v.at[k], sems[k])
            pltpu.async_copy(q_hbm.at[dstg_v.at[k]], q_v.at[k], sems[k])
            pltpu.async_copy(ce_hbm.at[eid_v.at[k]], ce_v.at[k], sems[k])

        def drain_compute_scatter(k):
            pltpu.make_async_copy(p_hbm.at[srcg_v.at[k]], p_v.at[k],
                                  sems[k]).wait()
            pltpu.make_async_copy(q_hbm.at[dstg_v.at[k]], q_v.at[k],
                                  sems[k]).wait()
            pltpu.make_async_copy(ce_hbm.at[eid_v.at[k]], ce_v.at[k],
                                  sems[k]).wait()

            def edge(j, _):
                for c in range(2):
                    o = c * 16
                    mp = (p_v[k, j, pl.ds(o, 16)] + q_v[k, j, pl.ds(o, 16)]
                          + ce_v[k, j, pl.ds(o, 16)])
                    sp = (p_v[k, j, pl.ds(_HD + o, 16)]
                          + q_v[k, j, pl.ds(_HD + o, 16)]
                          + ce_v[k, j, pl.ds(_HD + o, 16)])
                    sig = 1.0 / (1.0 + jnp.exp(jnp.minimum(-mp, 80.0)))
                    u = 1.0 + jnp.exp(jnp.maximum(-jnp.abs(sp), -80.0))
                    spl = jnp.maximum(sp, 0.0) + _log16(u)
                    msg_v[j, pl.ds(o, 16)] = sig * spl
                return 0

            lax.fori_loop(0, _B, edge, 0)
            pltpu.sync_copy(msg_v, acc.at[dsts_v.at[k]], add=True)

        load_and_fire(0, 0)

        def blk2(i, _):
            b0 = 2 * i
            load_and_fire(b0 + 1, 1)
            drain_compute_scatter(0)

            @pl.when(i < _NB // 2 - 1)
            def _():
                load_and_fire(b0 + 2, 0)

            drain_compute_scatter(1)
            return 0

        lax.fori_loop(0, _NB // 2, blk2, 0)
        plsc.subcore_barrier()
        pltpu.sync_copy(acc.at[pl.ds(sid * _RPT, _RPT)],
                        out_hbm.at[cid, half, pl.ds(sid * _RPT, _RPT)])
        plsc.subcore_barrier()


@functools.cache
def _sc_edge_kernel():
    return pl.kernel(
        _sc_edge_body,
        out_type=pltpu.MemorySpace.HBM((2, 2, _NP, _HD), jnp.float32),
        mesh=plsc.VectorSubcoreMesh(core_axis_name="c", subcore_axis_name="s"),
        scratch_types=[
            pltpu.VMEM((2, _B), jnp.int32),
            pltpu.VMEM((2, _B), jnp.int32),
            pltpu.VMEM((2, _B), jnp.int32),
            pltpu.VMEM((2, _B), jnp.int32),
            pltpu.VMEM((2, _B, _D), jnp.float32),
            pltpu.VMEM((2, _B, _D), jnp.float32),
            pltpu.VMEM((2, _B, _D), jnp.float32),
            pltpu.VMEM((_B, _HD), jnp.float32),
            pltpu.VMEM((_RPT, _HD), jnp.float32),
            pltpu.VMEM_SHARED((_NP, _HD), jnp.float32),
            pltpu.SemaphoreType.DMA,
            pltpu.SemaphoreType.DMA,
        ],
        compiler_params=pltpu.CompilerParams(use_tc_tiling_on_sc=False),
    )


def _sc_edge(*args):
    return _sc_edge_kernel()(*args)


def _embed_body(x_ref, wemb_ref, bemb_ref, wsrc_ref, wdst_ref,
                h_ref, pa_ref, pb_ref, qa_ref, qb_ref):
    z = _dot(x_ref[...], wemb_ref[0]) + bemb_ref[0]
    h = z * jax.nn.sigmoid(z)
    h_ref[...] = h
    psrc = _dot(h, wsrc_ref[0])
    qdst = _dot(h, wdst_ref[0])
    pa_ref[...] = psrc[:, :_D]
    pb_ref[...] = psrc[:, _D:]
    qa_ref[...] = qdst[:, :_D]
    qb_ref[...] = qdst[:, _D:]


def _eproj_body(e_ref, w_ref, b_ref, ca_ref, cb_ref):
    c = _dot(e_ref[...], w_ref[0, 0]) + b_ref[0, 0]
    ca_ref[...] = c[:, :_D]
    cb_ref[...] = c[:, _D:]


def _softplus_tc(z):
    return jnp.maximum(z, 0.0) + jnp.log1p(jnp.exp(-jnp.abs(z)))


def _nodeup_body(a0_ref, a1_ref, h_ref, bng_ref, bnb_ref, wsrc_ref, wdst_ref,
                 h2_ref, pa_ref, pb_ref, qa_ref, qb_ref):
    agg = jnp.concatenate([a0_ref[0, 0], a1_ref[0, 0]], axis=1)
    z = agg * bng_ref[0] + bnb_ref[0] + h_ref[...]
    h2 = _softplus_tc(z)
    h2_ref[...] = h2
    psrc = _dot(h2, wsrc_ref[0])
    qdst = _dot(h2, wdst_ref[0])
    pa_ref[...] = psrc[:, :_D]
    pb_ref[...] = psrc[:, _D:]
    qa_ref[...] = qdst[:, :_D]
    qb_ref[...] = qdst[:, _D:]


def _pool_body(h_ref, gid_ref, pooled_ref, counts_ref):
    i = pl.program_id(1)
    oh = (lax.broadcasted_iota(jnp.int32, (_NODE_BLK, _G), 1)
          == gid_ref[...]).astype(jnp.float32)
    ps = lax.dot_general(oh, h_ref[...], (((0,), (0,)), ((), ())),
                         precision=_HP)
    cs = jnp.sum(oh, axis=0)

    @pl.when(i == 0)
    def _():
        pooled_ref[...] = jnp.zeros_like(pooled_ref)
        counts_ref[...] = jnp.zeros_like(counts_ref)

    pooled_ref[...] += ps[None]
    counts_ref[...] += jnp.broadcast_to(cs[:, None], (_G, _D))[None]


def _head_body(pc_ref, cc_ref, w1_ref, b1_ref, w2_ref, b2_ref,
               wp_ref, bp_ref, o_ref):
    vs = pc_ref[0] / jnp.maximum(cc_ref[0], 1.0)
    vb = pc_ref[1] / jnp.maximum(cc_ref[1], 1.0)
    vt = jnp.concatenate([vs, vb], axis=1)
    z = _dot(vt, w1_ref[...]) + b1_ref[...]
    v = z * jax.nn.sigmoid(z)
    z = _dot(v, w2_ref[...]) + b2_ref[...]
    v = z * jax.nn.sigmoid(z)
    o_ref[...] = _dot(v, wp_ref[...]) + bp_ref[...]


_NODE_OUT_SPECS = [
    pl.BlockSpec((_NODE_BLK, _D), lambda c, i: (c * _NBLK + i, 0)),
    pl.BlockSpec((_NODE_BLK, _D), lambda c, i: (c * _NBLK + i, 0)),
    pl.BlockSpec((_NODE_BLK, _D), lambda c, i: (c * _NBLK + i, 0)),
    pl.BlockSpec((_NODE_BLK, _D), lambda c, i: (c * _NBLK + i, 0)),
    pl.BlockSpec((_NODE_BLK, _D), lambda c, i: (c * _NBLK + i, 0)),
]
_NODE_OUT_SHAPES = [
    jax.ShapeDtypeStruct((2 * _N, _D), jnp.float32),
    jax.ShapeDtypeStruct((2 * _N, _D), jnp.float32),
    jax.ShapeDtypeStruct((2 * _N, _D), jnp.float32),
    jax.ShapeDtypeStruct((2 * _N, _D), jnp.float32),
    jax.ShapeDtypeStruct((2 * _N, _D), jnp.float32),
]


def _embed_call(xcat, wemb2, bemb2, wsrc2, wdst2):
    return pl.pallas_call(
        _embed_body,
        grid=(2, _NBLK),
        in_specs=[
            pl.BlockSpec((_NODE_BLK, _DF), lambda c, i: (c * _NBLK + i, 0)),
            pl.BlockSpec((1, _DF, _D), lambda c, i: (c, 0, 0)),
            pl.BlockSpec((1, 1, _D), lambda c, i: (c, 0, 0)),
            pl.BlockSpec((1, _D, 2 * _D), lambda c, i: (c, 0, 0)),
            pl.BlockSpec((1, _D, 2 * _D), lambda c, i: (c, 0, 0)),
        ],
        out_specs=_NODE_OUT_SPECS,
        out_shape=_NODE_OUT_SHAPES,
    )(xcat, wemb2, bemb2, wsrc2, wdst2)


def _eproj_call(ecat, we23, be23):
    nblk = _EP // _EDGE_BLK
    return pl.pallas_call(
        _eproj_body,
        grid=(2, 3, nblk),
        in_specs=[
            pl.BlockSpec((_EDGE_BLK, _DE), lambda c, i, j: (c * nblk + j, 0)),
            pl.BlockSpec((1, 1, _DE, 2 * _D), lambda c, i, j: (c, i, 0, 0)),
            pl.BlockSpec((1, 1, 1, 2 * _D), lambda c, i, j: (c, i, 0, 0)),
        ],
        out_specs=[
            pl.BlockSpec((_EDGE_BLK, _D),
                         lambda c, i, j: ((c * 3 + i) * nblk + j, 0)),
            pl.BlockSpec((_EDGE_BLK, _D),
                         lambda c, i, j: ((c * 3 + i) * nblk + j, 0)),
        ],
        out_shape=[
            jax.ShapeDtypeStruct((6 * _EP, _D), jnp.float32),
            jax.ShapeDtypeStruct((6 * _EP, _D), jnp.float32),
        ],
    )(ecat, we23, be23)


def _nodeup_call(agg, hcat, bng2, bnb2, wsrc2, wdst2):
    return pl.pallas_call(
        _nodeup_body,
        grid=(2, _NBLK),
        in_specs=[
            pl.BlockSpec((1, 1, _NODE_BLK, _HD), lambda c, i: (c, 0, i, 0)),
            pl.BlockSpec((1, 1, _NODE_BLK, _HD), lambda c, i: (c, 1, i, 0)),
            pl.BlockSpec((_NODE_BLK, _D), lambda c, i: (c * _NBLK + i, 0)),
            pl.BlockSpec((1, 1, _D), lambda c, i: (c, 0, 0)),
            pl.BlockSpec((1, 1, _D), lambda c, i: (c, 0, 0)),
            pl.BlockSpec((1, _D, 2 * _D), lambda c, i: (c, 0, 0)),
            pl.BlockSpec((1, _D, 2 * _D), lambda c, i: (c, 0, 0)),
        ],
        out_specs=[
            pl.BlockSpec((_NODE_BLK, _D), lambda c, i: (c * _NBLK + i, 0)),
        ] + _NODE_OUT_SPECS[1:],
        out_shape=_NODE_OUT_SHAPES,
    )(agg, agg, hcat, bng2, bnb2, wsrc2, wdst2)


def _pool_call(hcat, gidcat):
    return pl.pallas_call(
        _pool_body,
        grid=(2, _NBLK),
        in_specs=[
            pl.BlockSpec((_NODE_BLK, _D), lambda c, i: (c * _NBLK + i, 0)),
            pl.BlockSpec((_NODE_BLK, 1), lambda c, i: (c * _NBLK + i, 0)),
        ],
        out_specs=[
            pl.BlockSpec((1, _G, _D), lambda c, i: (c, 0, 0)),
            pl.BlockSpec((1, _G, _D), lambda c, i: (c, 0, 0)),
        ],
        out_shape=[
            jax.ShapeDtypeStruct((2, _G, _D), jnp.float32),
            jax.ShapeDtypeStruct((2, _G, _D), jnp.float32),
        ],
    )(hcat, gidcat)


def _head_call(pooled, counts, w1, b1, w2, b2, wp, bp):
    return pl.pallas_call(
        _head_body,
        out_shape=jax.ShapeDtypeStruct((_G, 1), jnp.float32),
    )(pooled, counts, w1, b1[None, :], w2, b2[None, :], wp, bp[None, :])


_BN_S = float(1.0 / (1.0 + 1e-5) ** 0.5)


def _fold_lin(p):
    g = p['g'] * _BN_S
    return p['W'] * g[None, :], p['b'] * g + p['be']


def _split_cols(w1, w2):
    return jnp.concatenate([w1[:, :_HD], w2[:, :_HD],
                            w1[:, _HD:], w2[:, _HD:]], axis=1)


def _fold_conv(pc):
    w1, b1 = _fold_lin(pc['mlp'])
    w2, b2 = _fold_lin(pc['screen'])
    wsrc = _split_cols(w1[:_D], w2[:_D])
    wdst = _split_cols(w1[_D:2 * _D], w2[_D:2 * _D])
    we = _split_cols(w1[2 * _D:], w2[2 * _D:])
    be = jnp.concatenate([b1[:_HD], b2[:_HD], b1[_HD:], b2[_HD:]])
    return wsrc, wdst, we, be, pc['bn_g'] * _BN_S, pc['bn_b']


def _pad_idx(v, padval):
    return jnp.concatenate([v, jnp.full((_EP - _E,), padval, jnp.int32)])


def kernel(x_s, edge_index_s, e_s, graph_id_s, x_b, edge_index_b, e_b,
           graph_id_b, params):
    convs_s = [_fold_conv(pc) for pc in params['convs_s']]
    convs_b = [_fold_conv(pc) for pc in params['convs_b']]
    wemb_s, bemb_s = _fold_lin(params['emb_s'])
    wemb_b, bemb_b = _fold_lin(params['emb_b'])

    srcg2 = jnp.concatenate([_pad_idx(edge_index_s[0], 0),
                             _pad_idx(edge_index_b[0] + _N, _N)])
    dstg2 = jnp.concatenate([_pad_idx(edge_index_s[1], 0),
                             _pad_idx(edge_index_b[1] + _N, _N)])
    dsts2 = jnp.concatenate([_pad_idx(edge_index_s[1], _N),
                             _pad_idx(edge_index_b[1], _N)])

    zpad = jnp.zeros((_EP - _E, _DE), jnp.float32)
    ecat = jnp.concatenate([e_s, zpad, e_b, zpad])
    cea, ceb = _eproj_call(
        ecat,
        jnp.stack([jnp.stack([c[2] for c in convs_s]),
                   jnp.stack([c[2] for c in convs_b])]),
        jnp.stack([jnp.stack([c[3][None, :] for c in convs_s]),
                   jnp.stack([c[3][None, :] for c in convs_b])]),
    )

    xcat = jnp.concatenate([x_s, x_b])
    hcat, pa, pb, qa, qb = _embed_call(
        xcat,
        jnp.stack([wemb_s, wemb_b]),
        jnp.stack([bemb_s[None, :], bemb_b[None, :]]),
        jnp.stack([convs_s[0][0], convs_b[0][0]]),
        jnp.stack([convs_s[0][1], convs_b[0][1]]),
    )

    eids = jnp.arange(_EP, dtype=jnp.int32)
    xs = {
        'eid': jnp.stack([jnp.concatenate([eids + i * _EP,
                                           eids + (3 + i) * _EP])
                          for i in range(3)]),
        'bng': jnp.stack([jnp.stack([convs_s[i][4][None, :],
                                     convs_b[i][4][None, :]])
                          for i in range(3)]),
        'bnb': jnp.stack([jnp.stack([convs_s[i][5][None, :],
                                     convs_b[i][5][None, :]])
                          for i in range(3)]),
        'wsrc_n': jnp.stack([jnp.stack([convs_s[(i + 1) % 3][0],
                                        convs_b[(i + 1) % 3][0]])
                             for i in range(3)]),
        'wdst_n': jnp.stack([jnp.stack([convs_s[(i + 1) % 3][1],
                                        convs_b[(i + 1) % 3][1]])
                             for i in range(3)]),
    }

    def step(carry, x_i):
        h, pa_, pb_, qa_, qb_ = carry
        agg = _sc_edge(pa_, qa_, cea, pb_, qb_, ceb,
                       srcg2, dstg2, dsts2, x_i['eid'])
        h2, pa2, pb2, qa2, qb2 = _nodeup_call(agg, h, x_i['bng'], x_i['bnb'],
                                              x_i['wsrc_n'], x_i['wdst_n'])
        return (h2, pa2, pb2, qa2, qb2), None

    (hcat, _, _, _, _), _ = lax.scan(step, (hcat, pa, pb, qa, qb), xs)

    gidcat = jnp.concatenate([graph_id_s, graph_id_b])[:, None]
    pooled, counts = _pool_call(hcat, gidcat)

    w1, b1 = _fold_lin(params['fcs'][0])
    w2, b2 = _fold_lin(params['fcs'][1])
    return _head_call(pooled, counts, w1, b1, w2, b2,
                      params['pred']['W'], params['pred']['b'])
